# baseline (speedup 1.0000x reference)
import functools
import jax
import jax.numpy as jnp
from jax import lax
from jax.experimental import pallas as pl
from jax.experimental.pallas import tpu as pltpu
from jax.experimental.pallas import tpu_sc as plsc
from jax.experimental.layout import Format, Layout

B = 4096
SEQ = 200
D = 64
DP = 128
TOTAL = B * SEQ

NC = 2
NS = 16
NW = NC * NS
ROWS_PER_W = TOTAL // NW
SUB = 4
CHUNK = SUB * 128
NCH = ROWS_PER_W // CHUNK
BATCH_PER_W = B // NW


def _body(task0_hbm, uni_hbm, ttab_hbm, utab_hbm, out_hbm,
          idx_all, rows0, rows1, trow_v, didx_v, gs0, gs1, ss0, ss1):
    rows = (rows0, rows1)
    gs = (gs0, gs1)
    ss = (ss0, ss1)
    wid = lax.axis_index("s") * NC + lax.axis_index("c")
    r0 = wid * ROWS_PER_W

    pltpu.sync_copy(uni_hbm.at[pl.ds(r0, ROWS_PER_W)], idx_all)

    def fire_gather(b, c):
        for j in range(SUB):
            off = pl.multiple_of(c * CHUNK + j * 128, 128)
            pltpu.async_copy(
                utab_hbm.at[idx_all.at[pl.ds(off, 128)]],
                rows[b].at[pl.ds(j * 128, 128)], gs[b])

    def wait_gather(b):
        for j in range(SUB):
            pltpu.make_async_copy(
                utab_hbm.at[idx_all.at[pl.ds(j * 128, 128)]],
                rows[b].at[pl.ds(j * 128, 128)], gs[b]).wait()

    def fire_store(b, c):
        r = pl.multiple_of(r0 + c * CHUNK, CHUNK)
        pltpu.async_copy(rows[b],
                         out_hbm.at[pl.ds(r, CHUNK), pl.ds(0, D)], ss[b])

    def wait_store(b):
        pltpu.make_async_copy(
            rows[b], out_hbm.at[pl.ds(0, CHUNK), pl.ds(0, D)], ss[b]).wait()

    fire_gather(0, 0)
    fire_gather(1, 1)
    wait_gather(0)
    fire_store(0, 0)

    def pair(i, carry):
        for b in range(2):
            c = 2 * i + b
            wait_store(b)
            fire_gather(b, c)
            wait_gather(1 - b)
            fire_store(1 - b, c - 1)
        return carry

    lax.fori_loop(1, NCH // 2, pair, 0)

    wait_gather(1)
    fire_store(1, NCH - 1)
    wait_store(0)
    wait_store(1)

    b0 = wid * BATCH_PER_W
    pltpu.sync_copy(task0_hbm.at[pl.ds(b0, BATCH_PER_W)],
                    idx_all.at[pl.ds(0, BATCH_PER_W)])
    pltpu.async_copy(ttab_hbm.at[idx_all.at[pl.ds(0, BATCH_PER_W)]],
                     trow_v, gs0).wait()
    for j in range(BATCH_PER_W // 16):
        didx_v[pl.ds(j * 16, 16)] = (
            lax.iota(jnp.int32, 16) + (b0 + j * 16)) * SEQ
    pltpu.async_copy(trow_v, out_hbm.at[didx_v], gs0).wait()


_sc_call = functools.partial(
    pl.kernel,
    out_type=jax.ShapeDtypeStruct((TOTAL, DP), jnp.float32),
    mesh=plsc.VectorSubcoreMesh(core_axis_name="c", subcore_axis_name="s"),
    compiler_params=pltpu.CompilerParams(use_tc_tiling_on_sc=False),
    scratch_types=[
        pltpu.VMEM((ROWS_PER_W,), jnp.int32),
        pltpu.VMEM((CHUNK, D), jnp.float32),
        pltpu.VMEM((CHUNK, D), jnp.float32),
        pltpu.VMEM((BATCH_PER_W, DP), jnp.float32),
        pltpu.VMEM((BATCH_PER_W,), jnp.int32),
        pltpu.SemaphoreType.DMA,
        pltpu.SemaphoreType.DMA,
        pltpu.SemaphoreType.DMA,
        pltpu.SemaphoreType.DMA,
    ],
)(_body)


def _impl(task, uni, task_table, uni_table):
    task0 = task[:, 0].astype(jnp.int32)
    uni2 = uni.reshape(TOTAL).astype(jnp.int32) * 2
    utab_view = jnp.pad(uni_table, ((0, 0), (0, DP - D))).reshape(2 * 1000000, D)
    ttab_p = jnp.pad(task_table, ((0, 0), (0, DP - D)))
    out = _sc_call(task0, uni2, ttab_p, utab_view)
    return out[:, :D].reshape(B, SEQ, D)


_jit_cache = {}


def kernel(task, uni, task_table, uni_table):
    dev = jax.devices()[0]
    if dev.platform == "cpu":
        return _impl(task, uni, task_table, uni_table)
    fn = _jit_cache.get(dev)
    if fn is None:
        fmt = Format(Layout(major_to_minor=(0, 1, 2)),
                     jax.sharding.SingleDeviceSharding(dev))
        fn = jax.jit(_impl, out_shardings=fmt)
        _jit_cache[dev] = fn
    return fn(task, uni, task_table, uni_table)


# speedup vs baseline: 1.0020x; 1.0020x over previous
import functools
import jax
import jax.numpy as jnp
from jax import lax
from jax.experimental import pallas as pl
from jax.experimental.pallas import tpu as pltpu
from jax.experimental.pallas import tpu_sc as plsc

B = 4096
SEQ = 200
D = 64
DP = 128
TOTAL = B * SEQ

NC = 2
NS = 16
NW = NC * NS
ROWS_PER_W = TOTAL // NW
SUB = 4
CHUNK = SUB * 128
NCH = ROWS_PER_W // CHUNK
BATCH_PER_W = B // NW


def _body(task0_hbm, uni_hbm, ttab_hbm, utab_hbm, out_hbm,
          idx_all, rows0, rows1, trow_v, didx_v, gs0, gs1, ss0, ss1):
    rows = (rows0, rows1)
    gs = (gs0, gs1)
    ss = (ss0, ss1)
    wid = lax.axis_index("s") * NC + lax.axis_index("c")
    r0 = wid * ROWS_PER_W

    pltpu.sync_copy(uni_hbm.at[pl.ds(r0, ROWS_PER_W)], idx_all)

    def fire_gather(b, c):
        for j in range(SUB):
            off = pl.multiple_of(c * CHUNK + j * 128, 128)
            pltpu.async_copy(
                utab_hbm.at[idx_all.at[pl.ds(off, 128)]],
                rows[b].at[pl.ds(j * 128, 128)], gs[b])

    def wait_gather(b):
        for j in range(SUB):
            pltpu.make_async_copy(
                utab_hbm.at[idx_all.at[pl.ds(j * 128, 128)]],
                rows[b].at[pl.ds(j * 128, 128)], gs[b]).wait()

    def fire_store(b, c):
        r = pl.multiple_of(r0 + c * CHUNK, CHUNK)
        pltpu.async_copy(rows[b],
                         out_hbm.at[pl.ds(r, CHUNK), pl.ds(0, D)], ss[b])

    def wait_store(b):
        pltpu.make_async_copy(
            rows[b], out_hbm.at[pl.ds(0, CHUNK), pl.ds(0, D)], ss[b]).wait()

    fire_gather(0, 0)
    fire_gather(1, 1)
    wait_gather(0)
    fire_store(0, 0)

    def pair(i, carry):
        for b in range(2):
            c = 2 * i + b
            wait_store(b)
            fire_gather(b, c)
            wait_gather(1 - b)
            fire_store(1 - b, c - 1)
        return carry

    lax.fori_loop(1, NCH // 2, pair, 0)

    wait_gather(1)
    fire_store(1, NCH - 1)
    wait_store(0)
    wait_store(1)

    b0 = wid * BATCH_PER_W
    pltpu.sync_copy(task0_hbm.at[pl.ds(b0, BATCH_PER_W)],
                    idx_all.at[pl.ds(0, BATCH_PER_W)])
    pltpu.async_copy(ttab_hbm.at[idx_all.at[pl.ds(0, BATCH_PER_W)]],
                     trow_v, gs0).wait()
    for j in range(BATCH_PER_W // 16):
        didx_v[pl.ds(j * 16, 16)] = (
            lax.iota(jnp.int32, 16) + (b0 + j * 16)) * SEQ
    pltpu.async_copy(trow_v, out_hbm.at[didx_v], gs0).wait()


_sc_call = functools.partial(
    pl.kernel,
    out_type=jax.ShapeDtypeStruct((TOTAL, DP), jnp.float32),
    mesh=plsc.VectorSubcoreMesh(core_axis_name="c", subcore_axis_name="s"),
    compiler_params=pltpu.CompilerParams(use_tc_tiling_on_sc=False),
    scratch_types=[
        pltpu.VMEM((ROWS_PER_W,), jnp.int32),
        pltpu.VMEM((CHUNK, D), jnp.float32),
        pltpu.VMEM((CHUNK, D), jnp.float32),
        pltpu.VMEM((BATCH_PER_W, DP), jnp.float32),
        pltpu.VMEM((BATCH_PER_W,), jnp.int32),
        pltpu.SemaphoreType.DMA,
        pltpu.SemaphoreType.DMA,
        pltpu.SemaphoreType.DMA,
        pltpu.SemaphoreType.DMA,
    ],
)(_body)


@jax.jit
def kernel(task, uni, task_table, uni_table):
    task0 = task[:, 0].astype(jnp.int32)
    uni2 = uni.reshape(TOTAL).astype(jnp.int32) * 2
    utab_view = jnp.pad(uni_table, ((0, 0), (0, DP - D))).reshape(2 * 1000000, D)
    ttab_p = jnp.pad(task_table, ((0, 0), (0, DP - D)))
    out = _sc_call(task0, uni2, ttab_p, utab_view)
    return out[:, :D].reshape(B, SEQ, D)
